# Initial kernel scaffold; baseline (speedup 1.0000x reference)
import jax
import jax.numpy as jnp
from jax.experimental import pallas as pl


def kernel(sparse_inputs, tables):
    raise NotImplementedError("write your pallas kernel here")


# speedup vs baseline: 1.1445x; 1.1445x over previous
import functools

import jax
import jax.numpy as jnp
from jax import lax
from jax.experimental import pallas as pl
from jax.experimental.pallas import tpu as pltpu
from jax.experimental.pallas import tpu_sc as plsc

NUM_FIELDS = 26
VOCAB = 100000
DIM = 32
BATCH = 16384

NC = 2
NS = 16
NW = NC * NS
PER_W = BATCH // NW * NUM_FIELDS
GROUPS = PER_W // 16
IROW = 128
ROWS_I = PER_W // IROW
K = 8
NCHUNK = ROWS_I // K

_mesh = plsc.VectorSubcoreMesh(core_axis_name="c", subcore_axis_name="s")


@functools.partial(
    pl.kernel,
    out_type=jax.ShapeDtypeStruct((NW * ROWS_I, IROW, DIM), jnp.float32),
    mesh=_mesh,
    scratch_types=[
        pltpu.VMEM((ROWS_I, IROW), jnp.int32),
        pltpu.VMEM((K, IROW, DIM), jnp.float32),
        pltpu.SemaphoreType.DMA,
    ],
    compiler_params=pltpu.CompilerParams(use_tc_tiling_on_sc=False),
)
def _emb_lookup(idx_hbm, table_hbm, out_hbm, idx_v, rows_v, sem):
    wid = lax.axis_index("s") * NC + lax.axis_index("c")

    pltpu.sync_copy(idx_hbm.at[wid], idx_v)

    lanes = lax.iota(jnp.int32, 16)

    def _flatten(g, carry):
        row = g // (IROW // 16)
        col = (g % (IROW // 16)) * 16
        v = idx_v[row, pl.ds(col, 16)]
        p = g * 16 + lanes
        f = lax.rem(p, NUM_FIELDS)
        idx_v[row, pl.ds(col, 16)] = v + f * VOCAB
        return carry

    lax.fori_loop(0, GROUPS, _flatten, 0)

    def _chunk(c, carry):
        base = c * K
        cps = [
            pltpu.async_copy(table_hbm.at[idx_v.at[base + j]], rows_v.at[j], sem)
            for j in range(K)
        ]
        for cp in cps:
            cp.wait()
        pltpu.sync_copy(rows_v, out_hbm.at[pl.ds(wid * ROWS_I + base, K)])
        return carry

    lax.fori_loop(0, NCHUNK, _chunk, 0)


def kernel(sparse_inputs, tables):
    idx = sparse_inputs.astype(jnp.int32).reshape(NW, ROWS_I, IROW)
    table = tables.reshape(NUM_FIELDS * VOCAB, DIM)
    out = _emb_lookup(idx, table)
    return out.reshape(BATCH, NUM_FIELDS, DIM)


# speedup vs baseline: 4.0986x; 3.5812x over previous
import functools

import jax
import jax.numpy as jnp
from jax import lax
from jax.experimental import pallas as pl
from jax.experimental.pallas import tpu as pltpu
from jax.experimental.pallas import tpu_sc as plsc

NUM_FIELDS = 26
VOCAB = 100000
DIM = 32
BATCH = 16384

NC = 2
NS = 16
NW = NC * NS
PAIRS_PER_W = (NUM_FIELDS * DIM) // NW
HALF = BATCH // 2
G16 = HALF // 16

_mesh = plsc.VectorSubcoreMesh(core_axis_name="c", subcore_axis_name="s")


@functools.partial(
    pl.kernel,
    out_type=jax.ShapeDtypeStruct((NUM_FIELDS, DIM, BATCH), jnp.float32),
    mesh=_mesh,
    scratch_types=[
        pltpu.VMEM((BATCH,), jnp.int32),
        pltpu.VMEM((VOCAB,), jnp.float32),
        pltpu.VMEM((HALF,), jnp.float32),
    ],
    compiler_params=pltpu.CompilerParams(use_tc_tiling_on_sc=True, needs_layout_passes=False),
)
def _emb_lookup(idx_hbm, tab_hbm, out_hbm, idx_v, tab_v, out_v):
    wid = lax.axis_index("s") * NC + lax.axis_index("c")
    p0 = wid * PAIRS_PER_W

    def _pair(k, carry):
        p = p0 + k
        f = p // DIM
        d = lax.rem(p, DIM)

        @pl.when(jnp.logical_or(k == 0, d == 0))
        def _load_idx():
            pltpu.sync_copy(idx_hbm.at[f], idx_v)

        pltpu.sync_copy(tab_hbm.at[f, d], tab_v)

        def _half(h, c2):
            base = h * HALF

            def _g16(i, c3):
                idx16 = idx_v[pl.ds(base + i * 16, 16)]
                out_v[pl.ds(i * 16, 16)] = plsc.load_gather(tab_v, [idx16])
                return c3

            lax.fori_loop(0, G16, _g16, 0)
            pltpu.sync_copy(out_v, out_hbm.at[f, d, pl.ds(base, HALF)])
            return c2

        lax.fori_loop(0, 2, _half, 0)
        return carry

    lax.fori_loop(0, PAIRS_PER_W, _pair, 0)


def kernel(sparse_inputs, tables):
    idx_t = sparse_inputs.astype(jnp.int32).T
    tab_t = jnp.transpose(tables, (0, 2, 1))
    out_t = _emb_lookup(idx_t, tab_t)
    return jnp.transpose(out_t, (2, 0, 1))


# speedup vs baseline: 4.1841x; 1.0209x over previous
import functools

import jax
import jax.numpy as jnp
from jax import lax
from jax.experimental import pallas as pl
from jax.experimental.pallas import tpu as pltpu
from jax.experimental.pallas import tpu_sc as plsc

NUM_FIELDS = 26
VOCAB = 100000
DIM = 32
BATCH = 16384

NC = 2
NS = 16
NW = NC * NS
PAIRS_PER_W = (NUM_FIELDS * DIM) // NW
QOUT = BATCH // 4
G16 = QOUT // 16
NCH = 4
VCH = 25088
VTAIL = (VOCAB // 128) * 128
VCHS = [(0, VCH), (VCH, VCH), (2 * VCH, VCH), (3 * VCH, VTAIL - 3 * VCH)]
TAIL = VOCAB - VTAIL

_mesh = plsc.VectorSubcoreMesh(core_axis_name="c", subcore_axis_name="s")


@functools.partial(
    pl.kernel,
    out_type=jax.ShapeDtypeStruct((NUM_FIELDS, DIM, BATCH), jnp.float32),
    mesh=_mesh,
    scratch_types=[
        pltpu.VMEM((BATCH,), jnp.int32),
        pltpu.VMEM((VOCAB,), jnp.float32),
        pltpu.VMEM((2, QOUT), jnp.float32),
        pltpu.VMEM((DIM * TAIL,), jnp.float32),
        pltpu.SemaphoreType.DMA,
        pltpu.SemaphoreType.DMA,
        pltpu.SemaphoreType.DMA,
    ],
    compiler_params=pltpu.CompilerParams(use_tc_tiling_on_sc=True, needs_layout_passes=False),
)
def _emb_lookup(idx_hbm, tab_hbm, tails_hbm, out_hbm, idx_v, tab_v, out_v, tail_v,
                sem_t, sem_s0, sem_s1):
    wid = lax.axis_index("s") * NC + lax.axis_index("c")
    p0 = wid * PAIRS_PER_W
    sem_s = (sem_s0, sem_s1)

    def _store_drain(f, d, j):
        pltpu.make_async_copy(
            out_v.at[j % 2], out_hbm.at[f, d, pl.ds(j * QOUT, QOUT)], sem_s[j % 2]
        ).wait()

    def _pair(k, carry):
        p = p0 + k
        f = p // DIM
        d = lax.rem(p, DIM)

        tcps = [
            pltpu.async_copy(
                tab_hbm.at[f, d, pl.ds(off, sz)],
                tab_v.at[pl.ds(off, sz)],
                sem_t,
            )
            for off, sz in VCHS
        ]

        @pl.when(jnp.logical_or(k == 0, d == 0))
        def _load_idx():
            pltpu.sync_copy(idx_hbm.at[f], idx_v)
            pltpu.sync_copy(tails_hbm.at[f], tail_v)

        for t in range(TAIL // 16):
            tab_v[pl.ds(VTAIL + t * 16, 16)] = tail_v[pl.ds(d * TAIL + t * 16, 16)]

        for cp in tcps:
            cp.wait()

        for j in range(4):
            if j >= 2:
                _store_drain(f, d, j)
            else:
                @pl.when(k > 0)
                def _drain_prev():
                    _store_drain(f, d, j)
            base = j * QOUT

            def _g16(i, c3):
                idx16 = idx_v[pl.ds(base + i * 16, 16)]
                out_v[j % 2, pl.ds(i * 16, 16)] = plsc.load_gather(tab_v, [idx16])
                return c3

            lax.fori_loop(0, G16, _g16, 0, unroll=4)
            pltpu.async_copy(
                out_v.at[j % 2], out_hbm.at[f, d, pl.ds(base, QOUT)], sem_s[j % 2]
            )
        return carry

    lax.fori_loop(0, PAIRS_PER_W, _pair, 0)
    pl_last = p0 + PAIRS_PER_W - 1
    _store_drain(pl_last // DIM, lax.rem(pl_last, DIM), 2)
    _store_drain(pl_last // DIM, lax.rem(pl_last, DIM), 3)


def kernel(sparse_inputs, tables):
    idx_t = sparse_inputs.astype(jnp.int32).T
    tab_t = jnp.transpose(tables, (0, 2, 1))
    tails = jnp.transpose(tables[:, VTAIL:, :], (0, 2, 1)).reshape(NUM_FIELDS,
                                                                   DIM * TAIL)
    out_t = _emb_lookup(idx_t, tab_t, tails)
    return jnp.transpose(out_t, (2, 0, 1))
